# baseline (speedup 1.0000x reference)
import functools
import math

import jax
import jax.numpy as jnp
from jax import lax
from jax.experimental import pallas as pl
from jax.experimental.pallas import tpu as pltpu
from jax.experimental.pallas import tpu_sc as plsc

D_ = 64
SCALE_ = math.sqrt(D_)
NC_, NS_ = 2, 16
NW_ = NC_ * NS_
LAN_ = 16
BT_ = 128
T_ = 200
BJ_ = 32
CI_ = 8
NK_ = BT_ // LAN_


TCB_ = 1024


def _tc_transpose(tbl_t):
    n = tbl_t.shape[1]
    grid = (n + TCB_ - 1) // TCB_

    def body(in_ref, o_ref):
        blk = in_ref[...] * SCALE_
        o_ref[...] = (
            blk.reshape(D_, TCB_ // 2, 2)
            .transpose(1, 2, 0)
            .reshape(TCB_ // 2, 2 * D_)
        )

    return pl.pallas_call(
        body,
        grid=(grid,),
        in_specs=[pl.BlockSpec((D_, TCB_), lambda i: (0, i))],
        out_specs=pl.BlockSpec((TCB_ // 2, 2 * D_), lambda i: (i, 0)),
        out_shape=jax.ShapeDtypeStruct((500000, 2 * D_), jnp.float32),
    )(tbl_t)


def _make_emb():
    mesh = plsc.VectorSubcoreMesh(core_axis_name="c", subcore_axis_name="s")

    @functools.partial(
        pl.kernel,
        out_type=jax.ShapeDtypeStruct((T_, CI_, BJ_, 8, BT_), jnp.float32),
        mesh=mesh,
        scratch_types=[
            *[pltpu.VMEM((BT_,), jnp.int32) for _ in range(2)],
            *[pltpu.VMEM((2 * BT_,), jnp.int32) for _ in range(2)],
            *[pltpu.VMEM((2 * BT_, 32), jnp.float32) for _ in range(2)],
            *[pltpu.VMEM((D_, BT_), jnp.float32) for _ in range(2)],
            *[pltpu.SemaphoreType.DMA for _ in range(2)],
            *[pltpu.SemaphoreType.DMA for _ in range(2)],
            *[pltpu.SemaphoreType.DMA for _ in range(2)],
        ],
        compiler_params=pltpu.CompilerParams(
            use_tc_tiling_on_sc=False, needs_layout_passes=False
        ),
    )
    def emb(xt_hbm, tbl_hbm, out_hbm, *s):
        x_v = s[0:2]
        idx_v = s[2:4]
        rows_v = s[4:6]
        outt_v = s[6:8]
        sem_x = s[8:10]
        sem_g = s[10:12]
        sem_w = s[12:14]

        w = lax.axis_index("s") * NC_ + lax.axis_index("c")
        iota = lax.iota(jnp.int32, 16)
        toks = [iota + 16 * k for k in range(NK_)]
        rvec = [[2 * t + h for h in range(2)] for t in toks]
        pos_e = [2 * 16 * k + 2 * iota for k in range(NK_)]
        pos_o = [p + 1 for p in pos_e]

        def fire_x(t, q):
            pltpu.async_copy(xt_hbm.at[t * BJ_ + w], x_v[q], sem_x[q])

        def wait_x(q):
            pltpu.make_async_copy(xt_hbm.at[0], x_v[q], sem_x[q]).wait()

        def prep_idx(q):
            for k in range(NK_):
                xv2 = x_v[q][pl.ds(k * LAN_, LAN_)] * 2
                plsc.store_scatter(idx_v[q], [pos_e[k]], xv2)
                plsc.store_scatter(idx_v[q], [pos_o[k]], xv2 + 1)

        def fire_gather(p):
            for j in range(2):
                pltpu.async_copy(
                    tbl_hbm.at[idx_v[p].at[pl.ds(j * BT_, BT_)]],
                    rows_v[p].at[pl.ds(j * BT_, BT_)],
                    sem_g[p],
                )

        def wait_gather(p):
            for j in range(2):
                pltpu.make_async_copy(
                    tbl_hbm.at[pl.ds(0, BT_)],
                    rows_v[p].at[pl.ds(j * BT_, BT_)],
                    sem_g[p],
                ).wait()

        def transpose_scale(p):
            @plsc.parallel_loop(0, 16, step=1, unroll=2)
            def body(sft):
                diag = jnp.bitwise_and(iota + sft, 15)
                cvin = [diag, diag + 16]
                cvout = [diag, diag + 16, diag + 32, diag + 48]
                for k in range(NK_):
                    for cb in range(4):
                        v = plsc.load_gather(
                            rows_v[p], [rvec[k][cb >> 1], cvin[cb & 1]]
                        )
                        plsc.store_scatter(
                            outt_v[p], [cvout[cb], toks[k]], v
                        )

        def fire_writeout(t, p):
            for ci in range(CI_):
                pltpu.async_copy(
                    outt_v[p].at[pl.ds(ci * 8, 8)],
                    out_hbm.at[t, ci, w],
                    sem_w[p],
                )

        def wait_writeout(p):
            for ci in range(CI_):
                pltpu.make_async_copy(
                    outt_v[p].at[pl.ds(ci * 8, 8)],
                    out_hbm.at[0, 0, 0],
                    sem_w[p],
                ).wait()

        fire_x(0, 0)
        wait_x(0)
        prep_idx(0)
        fire_gather(0)
        fire_x(1, 1)

        def outer(TT, carry):
            for b2 in range(2):
                t = TT * 2 + b2
                p = b2
                q = 1 - b2

                wait_gather(p)

                @pl.when(t + 1 < T_)
                def _():
                    wait_x(q)

                    @pl.when(t + 2 < T_)
                    def _():
                        fire_x(t + 2, p)

                    prep_idx(q)
                    fire_gather(q)

                @pl.when(t >= 2)
                def _():
                    wait_writeout(p)

                transpose_scale(p)
                fire_writeout(t, p)
            return carry

        lax.fori_loop(0, T_ // 2, outer, 0)

        wait_writeout(0)
        wait_writeout(1)

    return emb


def kernel(x, table):
    xt = jnp.swapaxes(x, 0, 1).reshape(T_ * BJ_, BT_).astype(jnp.int32)
    tbl_rm = _tc_transpose(jnp.swapaxes(table, 0, 1))
    tbl = tbl_rm.reshape(2000000, 32)
    out5 = _make_emb()(xt, tbl)
    return out5.transpose(2, 4, 0, 1, 3).reshape(4096, 200, D_)


# speedup vs baseline: 5.8731x; 5.8731x over previous
import functools
import math

import jax
import jax.numpy as jnp
from jax import lax
from jax.experimental import pallas as pl
from jax.experimental.pallas import tpu as pltpu
from jax.experimental.pallas import tpu_sc as plsc

D_ = 64
SCALE_ = math.sqrt(D_)
NC_, NS_ = 2, 16
NW_ = NC_ * NS_
LAN_ = 16
BT_ = 128
T_ = 200
BJ_ = 32
CI_ = 8
NK_ = BT_ // LAN_


TCB_ = 512
TROWS_ = 500224


def _tc_transpose(tbl_t):
    n = tbl_t.shape[1]
    grid = 2 * (TROWS_ // TCB_)

    def body(in_ref, o_ref):
        h = pl.program_id(0) % 2
        data = jnp.transpose(in_ref[...]) * SCALE_

        @pl.when(h == 0)
        def _():
            o_ref[:, 0:D_] = data

        @pl.when(h == 1)
        def _():
            o_ref[:, D_:2 * D_] = data

    return pl.pallas_call(
        body,
        grid=(grid,),
        in_specs=[pl.BlockSpec((D_, TCB_), lambda i: (0, i))],
        out_specs=pl.BlockSpec((TCB_, 2 * D_), lambda i: (i // 2, 0)),
        out_shape=jax.ShapeDtypeStruct((TROWS_, 2 * D_), jnp.float32),
    )(tbl_t)


def _make_emb():
    mesh = plsc.VectorSubcoreMesh(core_axis_name="c", subcore_axis_name="s")

    @functools.partial(
        pl.kernel,
        out_type=jax.ShapeDtypeStruct((T_, CI_, BJ_, 8, BT_), jnp.float32),
        mesh=mesh,
        scratch_types=[
            *[pltpu.VMEM((BT_,), jnp.int32) for _ in range(2)],
            *[pltpu.VMEM((2 * BT_,), jnp.int32) for _ in range(2)],
            *[pltpu.VMEM((2 * BT_, 32), jnp.float32) for _ in range(2)],
            *[pltpu.VMEM((D_, BT_), jnp.float32) for _ in range(2)],
            *[pltpu.SemaphoreType.DMA for _ in range(2)],
            *[pltpu.SemaphoreType.DMA for _ in range(2)],
            *[pltpu.SemaphoreType.DMA for _ in range(2)],
        ],
        compiler_params=pltpu.CompilerParams(
            use_tc_tiling_on_sc=False, needs_layout_passes=False
        ),
    )
    def emb(xt_hbm, tbl_hbm, out_hbm, *s):
        x_v = s[0:2]
        idx_v = s[2:4]
        rows_v = s[4:6]
        outt_v = s[6:8]
        sem_x = s[8:10]
        sem_g = s[10:12]
        sem_w = s[12:14]

        w = lax.axis_index("s") * NC_ + lax.axis_index("c")
        iota = lax.iota(jnp.int32, 16)
        toks = [iota + 16 * k for k in range(NK_)]
        rvec = [[2 * t + h for h in range(2)] for t in toks]
        pos_e = [2 * 16 * k + 2 * iota for k in range(NK_)]
        pos_o = [p + 1 for p in pos_e]

        def fire_x(t, q):
            pltpu.async_copy(xt_hbm.at[t * BJ_ + w], x_v[q], sem_x[q])

        def wait_x(q):
            pltpu.make_async_copy(xt_hbm.at[0], x_v[q], sem_x[q]).wait()

        def prep_idx(q):
            for k in range(NK_):
                xv = x_v[q][pl.ds(k * LAN_, LAN_)]
                xv2 = (
                    lax.shift_left(lax.shift_right_logical(xv, 10), 11)
                    + lax.shift_left(jnp.bitwise_and(xv, 511), 2)
                    + lax.shift_left(
                        jnp.bitwise_and(lax.shift_right_logical(xv, 9), 1), 1
                    )
                )
                plsc.store_scatter(idx_v[q], [pos_e[k]], xv2)
                plsc.store_scatter(idx_v[q], [pos_o[k]], xv2 + 1)

        def fire_gather(p):
            for j in range(2):
                pltpu.async_copy(
                    tbl_hbm.at[idx_v[p].at[pl.ds(j * BT_, BT_)]],
                    rows_v[p].at[pl.ds(j * BT_, BT_)],
                    sem_g[p],
                )

        def wait_gather(p):
            for j in range(2):
                pltpu.make_async_copy(
                    tbl_hbm.at[pl.ds(0, BT_)],
                    rows_v[p].at[pl.ds(j * BT_, BT_)],
                    sem_g[p],
                ).wait()

        def transpose_scale(p):
            @plsc.parallel_loop(0, 16, step=1, unroll=2)
            def body(sft):
                diag = jnp.bitwise_and(iota + sft, 15)
                cvin = [diag, diag + 16]
                cvout = [diag, diag + 16, diag + 32, diag + 48]
                for k in range(NK_):
                    for cb in range(4):
                        v = plsc.load_gather(
                            rows_v[p], [rvec[k][cb >> 1], cvin[cb & 1]]
                        )
                        plsc.store_scatter(
                            outt_v[p], [cvout[cb], toks[k]], v
                        )

        def fire_writeout(t, p):
            for ci in range(CI_):
                pltpu.async_copy(
                    outt_v[p].at[pl.ds(ci * 8, 8)],
                    out_hbm.at[t, ci, w],
                    sem_w[p],
                )

        def wait_writeout(p):
            for ci in range(CI_):
                pltpu.make_async_copy(
                    outt_v[p].at[pl.ds(ci * 8, 8)],
                    out_hbm.at[0, 0, 0],
                    sem_w[p],
                ).wait()

        fire_x(0, 0)
        wait_x(0)
        prep_idx(0)
        fire_gather(0)
        fire_x(1, 1)

        def outer(TT, carry):
            for b2 in range(2):
                t = TT * 2 + b2
                p = b2
                q = 1 - b2

                wait_gather(p)

                @pl.when(t + 1 < T_)
                def _():
                    wait_x(q)

                    @pl.when(t + 2 < T_)
                    def _():
                        fire_x(t + 2, p)

                    prep_idx(q)
                    fire_gather(q)

                @pl.when(t >= 2)
                def _():
                    wait_writeout(p)

                transpose_scale(p)
                fire_writeout(t, p)
            return carry

        lax.fori_loop(0, T_ // 2, outer, 0)

        wait_writeout(0)
        wait_writeout(1)

    return emb


def kernel(x, table):
    xt = jnp.swapaxes(x, 0, 1).reshape(T_ * BJ_, BT_).astype(jnp.int32)
    tbl_rm = _tc_transpose(jnp.swapaxes(table, 0, 1))
    tbl = tbl_rm.reshape(4 * TROWS_, 32)
    out5 = _make_emb()(xt, tbl)
    return out5.transpose(2, 4, 0, 1, 3).reshape(4096, 200, D_)


# speedup vs baseline: 9.2873x; 1.5813x over previous
import functools
import math

import jax
import jax.numpy as jnp
from jax import lax
from jax.experimental import pallas as pl
from jax.experimental.pallas import tpu as pltpu
from jax.experimental.pallas import tpu_sc as plsc

D_ = 64
SCALE_ = math.sqrt(D_)
NC_, NS_ = 2, 16
NW_ = NC_ * NS_
LAN_ = 16
BT_ = 128
T_ = 200
BJ_ = 32
CI_ = 8
NK_ = BT_ // LAN_


TCB_ = 512
TROWS_ = 500224


def _tc_transpose(tbl_t):
    grid = TROWS_ // TCB_

    def body(in_ref, o_ref):
        blk = in_ref[...] * SCALE_
        o_ref[:, 0:D_] = jnp.transpose(blk[:, 0:TCB_])
        o_ref[:, D_:2 * D_] = jnp.transpose(blk[:, TCB_:2 * TCB_])

    return pl.pallas_call(
        body,
        grid=(grid,),
        in_specs=[pl.BlockSpec((D_, 2 * TCB_), lambda i: (0, i))],
        out_specs=pl.BlockSpec((TCB_, 2 * D_), lambda i: (i, 0)),
        out_shape=jax.ShapeDtypeStruct((TROWS_, 2 * D_), jnp.float32),
    )(tbl_t)


def _make_emb():
    mesh = plsc.VectorSubcoreMesh(core_axis_name="c", subcore_axis_name="s")

    @functools.partial(
        pl.kernel,
        out_type=jax.ShapeDtypeStruct((T_, CI_, BJ_, 8, BT_), jnp.float32),
        mesh=mesh,
        scratch_types=[
            *[pltpu.VMEM((BT_,), jnp.int32) for _ in range(2)],
            *[pltpu.VMEM((2 * BT_,), jnp.int32) for _ in range(2)],
            *[pltpu.VMEM((2 * BT_, 32), jnp.float32) for _ in range(2)],
            *[pltpu.VMEM((D_, BT_), jnp.float32) for _ in range(2)],
            *[pltpu.SemaphoreType.DMA for _ in range(2)],
            *[pltpu.SemaphoreType.DMA for _ in range(2)],
            *[pltpu.SemaphoreType.DMA for _ in range(2)],
        ],
        compiler_params=pltpu.CompilerParams(
            use_tc_tiling_on_sc=False, needs_layout_passes=False
        ),
    )
    def emb(xt_hbm, tbl_hbm, out_hbm, *s):
        x_v = s[0:2]
        idx_v = s[2:4]
        rows_v = s[4:6]
        outt_v = s[6:8]
        sem_x = s[8:10]
        sem_g = s[10:12]
        sem_w = s[12:14]

        w = lax.axis_index("s") * NC_ + lax.axis_index("c")
        iota = lax.iota(jnp.int32, 16)
        toks = [iota + 16 * k for k in range(NK_)]
        rvec = [[2 * t + h for h in range(2)] for t in toks]
        pos_e = [2 * 16 * k + 2 * iota for k in range(NK_)]
        pos_o = [p + 1 for p in pos_e]

        def fire_x(t, q):
            pltpu.async_copy(xt_hbm.at[t * BJ_ + w], x_v[q], sem_x[q])

        def wait_x(q):
            pltpu.make_async_copy(xt_hbm.at[0], x_v[q], sem_x[q]).wait()

        def prep_idx(q):
            for k in range(NK_):
                xv = x_v[q][pl.ds(k * LAN_, LAN_)]
                xv2 = (
                    lax.shift_left(lax.shift_right_logical(xv, 10), 11)
                    + lax.shift_left(jnp.bitwise_and(xv, 511), 2)
                    + lax.shift_left(
                        jnp.bitwise_and(lax.shift_right_logical(xv, 9), 1), 1
                    )
                )
                plsc.store_scatter(idx_v[q], [pos_e[k]], xv2)
                plsc.store_scatter(idx_v[q], [pos_o[k]], xv2 + 1)

        def fire_gather(p):
            for j in range(2):
                pltpu.async_copy(
                    tbl_hbm.at[idx_v[p].at[pl.ds(j * BT_, BT_)]],
                    rows_v[p].at[pl.ds(j * BT_, BT_)],
                    sem_g[p],
                )

        def wait_gather(p):
            for j in range(2):
                pltpu.make_async_copy(
                    tbl_hbm.at[pl.ds(0, BT_)],
                    rows_v[p].at[pl.ds(j * BT_, BT_)],
                    sem_g[p],
                ).wait()

        def transpose_scale(p):
            @plsc.parallel_loop(0, 16, step=1, unroll=2)
            def body(sft):
                diag = jnp.bitwise_and(iota + sft, 15)
                cvin = [diag, diag + 16]
                cvout = [diag, diag + 16, diag + 32, diag + 48]
                for k in range(NK_):
                    for cb in range(4):
                        v = plsc.load_gather(
                            rows_v[p], [rvec[k][cb >> 1], cvin[cb & 1]]
                        )
                        plsc.store_scatter(
                            outt_v[p], [cvout[cb], toks[k]], v
                        )

        def fire_writeout(t, p):
            for ci in range(CI_):
                pltpu.async_copy(
                    outt_v[p].at[pl.ds(ci * 8, 8)],
                    out_hbm.at[t, ci, w],
                    sem_w[p],
                )

        def wait_writeout(p):
            for ci in range(CI_):
                pltpu.make_async_copy(
                    outt_v[p].at[pl.ds(ci * 8, 8)],
                    out_hbm.at[0, 0, 0],
                    sem_w[p],
                ).wait()

        fire_x(0, 0)
        wait_x(0)
        prep_idx(0)
        fire_gather(0)
        fire_x(1, 1)

        def outer(TT, carry):
            for b2 in range(2):
                t = TT * 2 + b2
                p = b2
                q = 1 - b2

                wait_gather(p)

                @pl.when(t + 1 < T_)
                def _():
                    wait_x(q)

                    @pl.when(t + 2 < T_)
                    def _():
                        fire_x(t + 2, p)

                    prep_idx(q)
                    fire_gather(q)

                @pl.when(t >= 2)
                def _():
                    wait_writeout(p)

                transpose_scale(p)
                fire_writeout(t, p)
            return carry

        lax.fori_loop(0, T_ // 2, outer, 0)

        wait_writeout(0)
        wait_writeout(1)

    return emb


def kernel(x, table):
    xt = jnp.swapaxes(x, 0, 1).reshape(T_ * BJ_, BT_).astype(jnp.int32)
    tbl_rm = _tc_transpose(jnp.swapaxes(table, 0, 1))
    tbl = tbl_rm.reshape(4 * TROWS_, 32)
    out5 = _make_emb()(xt, tbl)
    return out5.transpose(2, 4, 0, 1, 3).reshape(4096, 200, D_)


# speedup vs baseline: 14.7935x; 1.5929x over previous
import functools
import math

import jax
import jax.numpy as jnp
from jax import lax
from jax.experimental import pallas as pl
from jax.experimental.pallas import tpu as pltpu
from jax.experimental.pallas import tpu_sc as plsc

D_ = 64
SCALE_ = math.sqrt(D_)
NC_, NS_ = 2, 16
NW_ = NC_ * NS_
LAN_ = 16
BT_ = 128
T_ = 200
BJ_ = 32
CI_ = 8
NK_ = BT_ // LAN_


TCB_ = 2048
SH_ = 11
TROWS_ = 245 * TCB_


def _tc_transpose(tbl_t):
    grid = TROWS_ // TCB_

    def body(in_ref, o_ref):
        blk = in_ref[...] * SCALE_
        o_ref[:, 0:D_] = jnp.transpose(blk[:, 0:TCB_])
        o_ref[:, D_:2 * D_] = jnp.transpose(blk[:, TCB_:2 * TCB_])

    return pl.pallas_call(
        body,
        grid=(grid,),
        in_specs=[pl.BlockSpec((D_, 2 * TCB_), lambda i: (0, i))],
        out_specs=pl.BlockSpec((TCB_, 2 * D_), lambda i: (i, 0)),
        out_shape=jax.ShapeDtypeStruct((TROWS_, 2 * D_), jnp.float32),
    )(tbl_t)


def _make_emb():
    mesh = plsc.VectorSubcoreMesh(core_axis_name="c", subcore_axis_name="s")

    @functools.partial(
        pl.kernel,
        out_type=jax.ShapeDtypeStruct((T_, CI_, BJ_, 8, BT_), jnp.float32),
        mesh=mesh,
        scratch_types=[
            *[pltpu.VMEM((BT_,), jnp.int32) for _ in range(2)],
            *[pltpu.VMEM((2 * BT_,), jnp.int32) for _ in range(2)],
            *[pltpu.VMEM((2 * BT_, 32), jnp.float32) for _ in range(2)],
            *[pltpu.VMEM((D_, BT_), jnp.float32) for _ in range(2)],
            *[pltpu.SemaphoreType.DMA for _ in range(2)],
            *[pltpu.SemaphoreType.DMA for _ in range(2)],
            *[pltpu.SemaphoreType.DMA for _ in range(2)],
        ],
        compiler_params=pltpu.CompilerParams(
            use_tc_tiling_on_sc=False, needs_layout_passes=False
        ),
    )
    def emb(xt_hbm, tbl_hbm, out_hbm, *s):
        x_v = s[0:2]
        idx_v = s[2:4]
        rows_v = s[4:6]
        outt_v = s[6:8]
        sem_x = s[8:10]
        sem_g = s[10:12]
        sem_w = s[12:14]

        w = lax.axis_index("s") * NC_ + lax.axis_index("c")
        iota = lax.iota(jnp.int32, 16)
        toks = [iota + 16 * k for k in range(NK_)]
        rvec = [[2 * t + h for h in range(2)] for t in toks]
        pos_e = [2 * 16 * k + 2 * iota for k in range(NK_)]
        pos_o = [p + 1 for p in pos_e]

        def fire_x(t, q):
            pltpu.async_copy(xt_hbm.at[t * BJ_ + w], x_v[q], sem_x[q])

        def wait_x(q):
            pltpu.make_async_copy(xt_hbm.at[0], x_v[q], sem_x[q]).wait()

        def prep_idx(q):
            for k in range(NK_):
                xv = x_v[q][pl.ds(k * LAN_, LAN_)]
                xv2 = (
                    lax.shift_left(lax.shift_right_logical(xv, SH_ + 1), SH_ + 2)
                    + lax.shift_left(jnp.bitwise_and(xv, TCB_ - 1), 2)
                    + lax.shift_left(
                        jnp.bitwise_and(lax.shift_right_logical(xv, SH_), 1), 1
                    )
                )
                plsc.store_scatter(idx_v[q], [pos_e[k]], xv2)
                plsc.store_scatter(idx_v[q], [pos_o[k]], xv2 + 1)

        def fire_gather(p):
            for j in range(2):
                pltpu.async_copy(
                    tbl_hbm.at[idx_v[p].at[pl.ds(j * BT_, BT_)]],
                    rows_v[p].at[pl.ds(j * BT_, BT_)],
                    sem_g[p],
                )

        def wait_gather(p):
            for j in range(2):
                pltpu.make_async_copy(
                    tbl_hbm.at[pl.ds(0, BT_)],
                    rows_v[p].at[pl.ds(j * BT_, BT_)],
                    sem_g[p],
                ).wait()

        def transpose_scale(p):
            @plsc.parallel_loop(0, 16, step=1, unroll=2)
            def body(sft):
                diag = jnp.bitwise_and(iota + sft, 15)
                cvin = [diag, diag + 16]
                cvout = [diag, diag + 16, diag + 32, diag + 48]
                for k in range(NK_):
                    for cb in range(4):
                        v = plsc.load_gather(
                            rows_v[p], [rvec[k][cb >> 1], cvin[cb & 1]]
                        )
                        plsc.store_scatter(
                            outt_v[p], [cvout[cb], toks[k]], v
                        )

        def fire_writeout(t, p):
            for ci in range(CI_):
                pltpu.async_copy(
                    outt_v[p].at[pl.ds(ci * 8, 8)],
                    out_hbm.at[t, ci, w],
                    sem_w[p],
                )

        def wait_writeout(p):
            for ci in range(CI_):
                pltpu.make_async_copy(
                    outt_v[p].at[pl.ds(ci * 8, 8)],
                    out_hbm.at[0, 0, 0],
                    sem_w[p],
                ).wait()

        fire_x(0, 0)
        wait_x(0)
        prep_idx(0)
        fire_gather(0)
        fire_x(1, 1)

        def outer(TT, carry):
            for b2 in range(2):
                t = TT * 2 + b2
                p = b2
                q = 1 - b2

                wait_gather(p)

                @pl.when(t + 1 < T_)
                def _():
                    wait_x(q)

                    @pl.when(t + 2 < T_)
                    def _():
                        fire_x(t + 2, p)

                    prep_idx(q)
                    fire_gather(q)

                @pl.when(t >= 2)
                def _():
                    wait_writeout(p)

                transpose_scale(p)
                fire_writeout(t, p)
            return carry

        lax.fori_loop(0, T_ // 2, outer, 0)

        wait_writeout(0)
        wait_writeout(1)

    return emb


def kernel(x, table):
    xt = jnp.swapaxes(x, 0, 1).reshape(T_ * BJ_, BT_).astype(jnp.int32)
    tbl_rm = _tc_transpose(jnp.swapaxes(table, 0, 1))
    tbl = tbl_rm.reshape(4 * TROWS_, 32)
    out5 = _make_emb()(xt, tbl)
    return out5.transpose(2, 4, 0, 1, 3).reshape(4096, 200, D_)


# speedup vs baseline: 16.6530x; 1.1257x over previous
import functools
import math

import jax
import jax.numpy as jnp
from jax import lax
from jax.experimental import pallas as pl
from jax.experimental.pallas import tpu as pltpu
from jax.experimental.pallas import tpu_sc as plsc

D_ = 64
SCALE_ = math.sqrt(D_)
NC_, NS_ = 2, 16
NW_ = NC_ * NS_
LAN_ = 16
BT_ = 128
T_ = 200
BJ_ = 32
CI_ = 8
NK_ = BT_ // LAN_


TCB_ = 4096
SH_ = 12
TROWS_ = 123 * TCB_


def _tc_transpose(tbl_t):
    grid = TROWS_ // TCB_

    def body(in_ref, o_ref):
        blk = in_ref[...] * SCALE_
        o_ref[:, 0:D_] = jnp.transpose(blk[:, 0:TCB_])
        o_ref[:, D_:2 * D_] = jnp.transpose(blk[:, TCB_:2 * TCB_])

    return pl.pallas_call(
        body,
        grid=(grid,),
        in_specs=[pl.BlockSpec((D_, 2 * TCB_), lambda i: (0, i))],
        out_specs=pl.BlockSpec((TCB_, 2 * D_), lambda i: (i, 0)),
        out_shape=jax.ShapeDtypeStruct((TROWS_, 2 * D_), jnp.float32),
    )(tbl_t)


def _make_emb():
    mesh = plsc.VectorSubcoreMesh(core_axis_name="c", subcore_axis_name="s")

    @functools.partial(
        pl.kernel,
        out_type=jax.ShapeDtypeStruct((T_, CI_, BJ_, 8, BT_), jnp.float32),
        mesh=mesh,
        scratch_types=[
            *[pltpu.VMEM((BT_,), jnp.int32) for _ in range(2)],
            *[pltpu.VMEM((2 * BT_,), jnp.int32) for _ in range(2)],
            *[pltpu.VMEM((2 * BT_, 32), jnp.float32) for _ in range(2)],
            *[pltpu.VMEM((D_, BT_), jnp.float32) for _ in range(2)],
            *[pltpu.SemaphoreType.DMA for _ in range(2)],
            *[pltpu.SemaphoreType.DMA for _ in range(2)],
            *[pltpu.SemaphoreType.DMA for _ in range(2)],
        ],
        compiler_params=pltpu.CompilerParams(
            use_tc_tiling_on_sc=False, needs_layout_passes=False
        ),
    )
    def emb(xt_hbm, tbl_hbm, out_hbm, *s):
        x_v = s[0:2]
        idx_v = s[2:4]
        rows_v = s[4:6]
        outt_v = s[6:8]
        sem_x = s[8:10]
        sem_g = s[10:12]
        sem_w = s[12:14]

        w = lax.axis_index("s") * NC_ + lax.axis_index("c")
        iota = lax.iota(jnp.int32, 16)
        toks = [iota + 16 * k for k in range(NK_)]
        rvec = [[2 * t + h for h in range(2)] for t in toks]
        pos_e = [2 * 16 * k + 2 * iota for k in range(NK_)]
        pos_o = [p + 1 for p in pos_e]

        def fire_x(t, q):
            pltpu.async_copy(xt_hbm.at[t * BJ_ + w], x_v[q], sem_x[q])

        def wait_x(q):
            pltpu.make_async_copy(xt_hbm.at[0], x_v[q], sem_x[q]).wait()

        def prep_idx(q):
            for k in range(NK_):
                xv = x_v[q][pl.ds(k * LAN_, LAN_)]
                xv2 = (
                    lax.shift_left(lax.shift_right_logical(xv, SH_ + 1), SH_ + 2)
                    + lax.shift_left(jnp.bitwise_and(xv, TCB_ - 1), 2)
                    + lax.shift_left(
                        jnp.bitwise_and(lax.shift_right_logical(xv, SH_), 1), 1
                    )
                )
                plsc.store_scatter(idx_v[q], [pos_e[k]], xv2)
                plsc.store_scatter(idx_v[q], [pos_o[k]], xv2 + 1)

        def fire_gather(p):
            for j in range(2):
                pltpu.async_copy(
                    tbl_hbm.at[idx_v[p].at[pl.ds(j * BT_, BT_)]],
                    rows_v[p].at[pl.ds(j * BT_, BT_)],
                    sem_g[p],
                )

        def wait_gather(p):
            for j in range(2):
                pltpu.make_async_copy(
                    tbl_hbm.at[pl.ds(0, BT_)],
                    rows_v[p].at[pl.ds(j * BT_, BT_)],
                    sem_g[p],
                ).wait()

        def transpose_scale(p):
            @plsc.parallel_loop(0, 16, step=1, unroll=2)
            def body(sft):
                diag = jnp.bitwise_and(iota + sft, 15)
                cvin = [diag, diag + 16]
                cvout = [diag, diag + 16, diag + 32, diag + 48]
                for k in range(NK_):
                    for cb in range(4):
                        v = plsc.load_gather(
                            rows_v[p], [rvec[k][cb >> 1], cvin[cb & 1]]
                        )
                        plsc.store_scatter(
                            outt_v[p], [cvout[cb], toks[k]], v
                        )

        def fire_writeout(t, p):
            for ci in range(CI_):
                pltpu.async_copy(
                    outt_v[p].at[pl.ds(ci * 8, 8)],
                    out_hbm.at[t, ci, w],
                    sem_w[p],
                )

        def wait_writeout(p):
            for ci in range(CI_):
                pltpu.make_async_copy(
                    outt_v[p].at[pl.ds(ci * 8, 8)],
                    out_hbm.at[0, 0, 0],
                    sem_w[p],
                ).wait()

        fire_x(0, 0)
        wait_x(0)
        prep_idx(0)
        fire_gather(0)
        fire_x(1, 1)

        def outer(TT, carry):
            for b2 in range(2):
                t = TT * 2 + b2
                p = b2
                q = 1 - b2

                wait_gather(p)

                @pl.when(t + 1 < T_)
                def _():
                    wait_x(q)

                    @pl.when(t + 2 < T_)
                    def _():
                        fire_x(t + 2, p)

                    prep_idx(q)
                    fire_gather(q)

                @pl.when(t >= 2)
                def _():
                    wait_writeout(p)

                transpose_scale(p)
                fire_writeout(t, p)
            return carry

        lax.fori_loop(0, T_ // 2, outer, 0)

        wait_writeout(0)
        wait_writeout(1)

    return emb


def kernel(x, table):
    xt = jnp.swapaxes(x, 0, 1).reshape(T_ * BJ_, BT_).astype(jnp.int32)
    tbl_rm = _tc_transpose(jnp.swapaxes(table, 0, 1))
    tbl = tbl_rm.reshape(4 * TROWS_, 32)
    out5 = _make_emb()(xt, tbl)
    return out5.transpose(2, 4, 0, 1, 3).reshape(4096, 200, D_)


# speedup vs baseline: 17.7211x; 1.0641x over previous
import functools
import math

import jax
import jax.numpy as jnp
from jax import lax
from jax.experimental import pallas as pl
from jax.experimental.pallas import tpu as pltpu
from jax.experimental.pallas import tpu_sc as plsc

D_ = 64
SCALE_ = math.sqrt(D_)
NC_, NS_ = 2, 16
NW_ = NC_ * NS_
LAN_ = 16
BT_ = 128
T_ = 200
BJ_ = 32
CI_ = 8
NK_ = BT_ // LAN_


TCB_ = 8192
SH_ = 13
TROWS_ = 62 * TCB_


def _tc_transpose(tbl_t):
    grid = TROWS_ // TCB_

    def body(in_ref, o_ref):
        blk = in_ref[...] * SCALE_
        o_ref[:, 0:D_] = jnp.transpose(blk[:, 0:TCB_])
        o_ref[:, D_:2 * D_] = jnp.transpose(blk[:, TCB_:2 * TCB_])

    return pl.pallas_call(
        body,
        grid=(grid,),
        in_specs=[pl.BlockSpec((D_, 2 * TCB_), lambda i: (0, i))],
        out_specs=pl.BlockSpec((TCB_, 2 * D_), lambda i: (i, 0)),
        out_shape=jax.ShapeDtypeStruct((TROWS_, 2 * D_), jnp.float32),
    )(tbl_t)


def _make_emb():
    mesh = plsc.VectorSubcoreMesh(core_axis_name="c", subcore_axis_name="s")

    @functools.partial(
        pl.kernel,
        out_type=jax.ShapeDtypeStruct((T_, CI_, BJ_, 8, BT_), jnp.float32),
        mesh=mesh,
        scratch_types=[
            *[pltpu.VMEM((BT_,), jnp.int32) for _ in range(2)],
            *[pltpu.VMEM((2 * BT_,), jnp.int32) for _ in range(2)],
            *[pltpu.VMEM((2 * BT_, 32), jnp.float32) for _ in range(2)],
            *[pltpu.VMEM((D_, BT_), jnp.float32) for _ in range(2)],
            *[pltpu.SemaphoreType.DMA for _ in range(2)],
            *[pltpu.SemaphoreType.DMA for _ in range(2)],
            *[pltpu.SemaphoreType.DMA for _ in range(2)],
        ],
        compiler_params=pltpu.CompilerParams(
            use_tc_tiling_on_sc=False, needs_layout_passes=False
        ),
    )
    def emb(xt_hbm, tbl_hbm, out_hbm, *s):
        x_v = s[0:2]
        idx_v = s[2:4]
        rows_v = s[4:6]
        outt_v = s[6:8]
        sem_x = s[8:10]
        sem_g = s[10:12]
        sem_w = s[12:14]

        w = lax.axis_index("s") * NC_ + lax.axis_index("c")
        iota = lax.iota(jnp.int32, 16)
        toks = [iota + 16 * k for k in range(NK_)]
        rvec = [[2 * t + h for h in range(2)] for t in toks]
        pos_e = [2 * 16 * k + 2 * iota for k in range(NK_)]
        pos_o = [p + 1 for p in pos_e]

        def fire_x(t, q):
            pltpu.async_copy(xt_hbm.at[t * BJ_ + w], x_v[q], sem_x[q])

        def wait_x(q):
            pltpu.make_async_copy(xt_hbm.at[0], x_v[q], sem_x[q]).wait()

        def prep_idx(q):
            for k in range(NK_):
                xv = x_v[q][pl.ds(k * LAN_, LAN_)]
                xv2 = (
                    lax.shift_left(lax.shift_right_logical(xv, SH_ + 1), SH_ + 2)
                    + lax.shift_left(jnp.bitwise_and(xv, TCB_ - 1), 2)
                    + lax.shift_left(
                        jnp.bitwise_and(lax.shift_right_logical(xv, SH_), 1), 1
                    )
                )
                plsc.store_scatter(idx_v[q], [pos_e[k]], xv2)
                plsc.store_scatter(idx_v[q], [pos_o[k]], xv2 + 1)

        def fire_gather(p):
            for j in range(2):
                pltpu.async_copy(
                    tbl_hbm.at[idx_v[p].at[pl.ds(j * BT_, BT_)]],
                    rows_v[p].at[pl.ds(j * BT_, BT_)],
                    sem_g[p],
                )

        def wait_gather(p):
            for j in range(2):
                pltpu.make_async_copy(
                    tbl_hbm.at[pl.ds(0, BT_)],
                    rows_v[p].at[pl.ds(j * BT_, BT_)],
                    sem_g[p],
                ).wait()

        def transpose_scale(p):
            @plsc.parallel_loop(0, 16, step=1, unroll=2)
            def body(sft):
                diag = jnp.bitwise_and(iota + sft, 15)
                cvin = [diag, diag + 16]
                cvout = [diag, diag + 16, diag + 32, diag + 48]
                for k in range(NK_):
                    for cb in range(4):
                        v = plsc.load_gather(
                            rows_v[p], [rvec[k][cb >> 1], cvin[cb & 1]]
                        )
                        plsc.store_scatter(
                            outt_v[p], [cvout[cb], toks[k]], v
                        )

        def fire_writeout(t, p):
            for ci in range(CI_):
                pltpu.async_copy(
                    outt_v[p].at[pl.ds(ci * 8, 8)],
                    out_hbm.at[t, ci, w],
                    sem_w[p],
                )

        def wait_writeout(p):
            for ci in range(CI_):
                pltpu.make_async_copy(
                    outt_v[p].at[pl.ds(ci * 8, 8)],
                    out_hbm.at[0, 0, 0],
                    sem_w[p],
                ).wait()

        fire_x(0, 0)
        wait_x(0)
        prep_idx(0)
        fire_gather(0)
        fire_x(1, 1)

        def outer(TT, carry):
            for b2 in range(2):
                t = TT * 2 + b2
                p = b2
                q = 1 - b2

                wait_gather(p)

                @pl.when(t + 1 < T_)
                def _():
                    wait_x(q)

                    @pl.when(t + 2 < T_)
                    def _():
                        fire_x(t + 2, p)

                    prep_idx(q)
                    fire_gather(q)

                @pl.when(t >= 2)
                def _():
                    wait_writeout(p)

                transpose_scale(p)
                fire_writeout(t, p)
            return carry

        lax.fori_loop(0, T_ // 2, outer, 0)

        wait_writeout(0)
        wait_writeout(1)

    return emb


def kernel(x, table):
    xt = jnp.swapaxes(x, 0, 1).reshape(T_ * BJ_, BT_).astype(jnp.int32)
    tbl_rm = _tc_transpose(jnp.swapaxes(table, 0, 1))
    tbl = tbl_rm.reshape(4 * TROWS_, 32)
    out5 = _make_emb()(xt, tbl)
    return out5.transpose(2, 4, 0, 1, 3).reshape(4096, 200, D_)


# speedup vs baseline: 18.2384x; 1.0292x over previous
import functools
import math

import jax
import jax.numpy as jnp
from jax import lax
from jax.experimental import pallas as pl
from jax.experimental.pallas import tpu as pltpu
from jax.experimental.pallas import tpu_sc as plsc

D_ = 64
SCALE_ = math.sqrt(D_)
NC_, NS_ = 2, 16
NW_ = NC_ * NS_
LAN_ = 16
BT_ = 128
T_ = 200
BJ_ = 32
CI_ = 8
NK_ = BT_ // LAN_


TCB_ = 16384
SH_ = 14
TROWS_ = 31 * TCB_


def _tc_transpose(tbl_t):
    grid = TROWS_ // TCB_

    def body(in_ref, o_ref):
        blk = in_ref[...] * SCALE_
        o_ref[:, 0:D_] = jnp.transpose(blk[:, 0:TCB_])
        o_ref[:, D_:2 * D_] = jnp.transpose(blk[:, TCB_:2 * TCB_])

    return pl.pallas_call(
        body,
        grid=(grid,),
        in_specs=[pl.BlockSpec((D_, 2 * TCB_), lambda i: (0, i))],
        out_specs=pl.BlockSpec((TCB_, 2 * D_), lambda i: (i, 0)),
        out_shape=jax.ShapeDtypeStruct((TROWS_, 2 * D_), jnp.float32),
    )(tbl_t)


def _make_emb():
    mesh = plsc.VectorSubcoreMesh(core_axis_name="c", subcore_axis_name="s")

    @functools.partial(
        pl.kernel,
        out_type=jax.ShapeDtypeStruct((T_, CI_, BJ_, 8, BT_), jnp.float32),
        mesh=mesh,
        scratch_types=[
            *[pltpu.VMEM((BT_,), jnp.int32) for _ in range(2)],
            *[pltpu.VMEM((2 * BT_,), jnp.int32) for _ in range(2)],
            *[pltpu.VMEM((2 * BT_, 32), jnp.float32) for _ in range(2)],
            *[pltpu.VMEM((D_, BT_), jnp.float32) for _ in range(2)],
            *[pltpu.SemaphoreType.DMA for _ in range(2)],
            *[pltpu.SemaphoreType.DMA for _ in range(2)],
            *[pltpu.SemaphoreType.DMA for _ in range(2)],
        ],
        compiler_params=pltpu.CompilerParams(
            use_tc_tiling_on_sc=False, needs_layout_passes=False
        ),
    )
    def emb(xt_hbm, tbl_hbm, out_hbm, *s):
        x_v = s[0:2]
        idx_v = s[2:4]
        rows_v = s[4:6]
        outt_v = s[6:8]
        sem_x = s[8:10]
        sem_g = s[10:12]
        sem_w = s[12:14]

        w = lax.axis_index("s") * NC_ + lax.axis_index("c")
        iota = lax.iota(jnp.int32, 16)
        toks = [iota + 16 * k for k in range(NK_)]
        rvec = [[2 * t + h for h in range(2)] for t in toks]
        pos_e = [2 * 16 * k + 2 * iota for k in range(NK_)]
        pos_o = [p + 1 for p in pos_e]

        def fire_x(t, q):
            pltpu.async_copy(xt_hbm.at[t * BJ_ + w], x_v[q], sem_x[q])

        def wait_x(q):
            pltpu.make_async_copy(xt_hbm.at[0], x_v[q], sem_x[q]).wait()

        def prep_idx(q):
            for k in range(NK_):
                xv = x_v[q][pl.ds(k * LAN_, LAN_)]
                xv2 = (
                    lax.shift_left(lax.shift_right_logical(xv, SH_ + 1), SH_ + 2)
                    + lax.shift_left(jnp.bitwise_and(xv, TCB_ - 1), 2)
                    + lax.shift_left(
                        jnp.bitwise_and(lax.shift_right_logical(xv, SH_), 1), 1
                    )
                )
                plsc.store_scatter(idx_v[q], [pos_e[k]], xv2)
                plsc.store_scatter(idx_v[q], [pos_o[k]], xv2 + 1)

        def fire_gather(p):
            for j in range(2):
                pltpu.async_copy(
                    tbl_hbm.at[idx_v[p].at[pl.ds(j * BT_, BT_)]],
                    rows_v[p].at[pl.ds(j * BT_, BT_)],
                    sem_g[p],
                )

        def wait_gather(p):
            for j in range(2):
                pltpu.make_async_copy(
                    tbl_hbm.at[pl.ds(0, BT_)],
                    rows_v[p].at[pl.ds(j * BT_, BT_)],
                    sem_g[p],
                ).wait()

        def transpose_scale(p):
            @plsc.parallel_loop(0, 16, step=1, unroll=2)
            def body(sft):
                diag = jnp.bitwise_and(iota + sft, 15)
                cvin = [diag, diag + 16]
                cvout = [diag, diag + 16, diag + 32, diag + 48]
                for k in range(NK_):
                    for cb in range(4):
                        v = plsc.load_gather(
                            rows_v[p], [rvec[k][cb >> 1], cvin[cb & 1]]
                        )
                        plsc.store_scatter(
                            outt_v[p], [cvout[cb], toks[k]], v
                        )

        def fire_writeout(t, p):
            for ci in range(CI_):
                pltpu.async_copy(
                    outt_v[p].at[pl.ds(ci * 8, 8)],
                    out_hbm.at[t, ci, w],
                    sem_w[p],
                )

        def wait_writeout(p):
            for ci in range(CI_):
                pltpu.make_async_copy(
                    outt_v[p].at[pl.ds(ci * 8, 8)],
                    out_hbm.at[0, 0, 0],
                    sem_w[p],
                ).wait()

        fire_x(0, 0)
        wait_x(0)
        prep_idx(0)
        fire_gather(0)
        fire_x(1, 1)

        def outer(TT, carry):
            for b2 in range(2):
                t = TT * 2 + b2
                p = b2
                q = 1 - b2

                wait_gather(p)

                @pl.when(t + 1 < T_)
                def _():
                    wait_x(q)

                    @pl.when(t + 2 < T_)
                    def _():
                        fire_x(t + 2, p)

                    prep_idx(q)
                    fire_gather(q)

                @pl.when(t >= 2)
                def _():
                    wait_writeout(p)

                transpose_scale(p)
                fire_writeout(t, p)
            return carry

        lax.fori_loop(0, T_ // 2, outer, 0)

        wait_writeout(0)
        wait_writeout(1)

    return emb


def kernel(x, table):
    xt = jnp.swapaxes(x, 0, 1).reshape(T_ * BJ_, BT_).astype(jnp.int32)
    tbl_rm = _tc_transpose(jnp.swapaxes(table, 0, 1))
    tbl = tbl_rm.reshape(4 * TROWS_, 32)
    out5 = _make_emb()(xt, tbl)
    return out5.transpose(2, 4, 0, 1, 3).reshape(4096, 200, D_)
